# initial kernel scaffold (unmeasured)
import jax
import jax.numpy as jnp
from jax import lax
from jax.experimental import pallas as pl
from jax.experimental.pallas import tpu as pltpu

N_DEV = 32
BLK = 128


def kernel(x, w_mat):
    k, m_per = x.shape
    _, n = w_mat.shape

    def body(x_ref, w_ref, out_ref, xg_ref, y_ref, amax_ref,
             send_sems, recv_sems, asend_sems, arecv_sems):
        my = lax.axis_index("i")

        for j in range(N_DEV):
            @pl.when(my != j)
            def _(j=j):
                rdma = pltpu.make_async_remote_copy(
                    src_ref=x_ref.at[pl.ds(j * BLK, BLK), :],
                    dst_ref=xg_ref.at[:, pl.ds(my * BLK, BLK)],
                    send_sem=send_sems.at[j],
                    recv_sem=recv_sems.at[my],
                    device_id=(j,),
                    device_id_type=pl.DeviceIdType.MESH,
                )
                rdma.start()

        xg_ref[:, pl.ds(my * BLK, BLK)] = x_ref[pl.ds(my * BLK, BLK), :]

        for j in range(N_DEV):
            @pl.when(my != j)
            def _(j=j):
                recv = pltpu.make_async_remote_copy(
                    src_ref=x_ref.at[pl.ds(0, BLK), :],
                    dst_ref=xg_ref.at[:, pl.ds(j * BLK, BLK)],
                    send_sem=send_sems.at[j],
                    recv_sem=recv_sems.at[j],
                    device_id=(j,),
                    device_id_type=pl.DeviceIdType.MESH,
                )
                recv.wait_recv()

        y = jnp.dot(
            xg_ref[:, :], w_ref[:, :],
            preferred_element_type=jnp.float32,
            precision=lax.Precision.HIGHEST,
        )
        y = jnp.maximum(y, 0.0)
        y_ref[:, :] = y
        amax_ref[my, :, :] = jnp.full((8, 128), jnp.max(y), jnp.float32)

        for j in range(N_DEV):
            @pl.when(my != j)
            def _(j=j):
                rdma = pltpu.make_async_remote_copy(
                    src_ref=amax_ref.at[my],
                    dst_ref=amax_ref.at[my],
                    send_sem=asend_sems.at[j],
                    recv_sem=arecv_sems.at[my],
                    device_id=(j,),
                    device_id_type=pl.DeviceIdType.MESH,
                )
                rdma.start()
        for j in range(N_DEV):
            @pl.when(my != j)
            def _(j=j):
                recv = pltpu.make_async_remote_copy(
                    src_ref=amax_ref.at[my],
                    dst_ref=amax_ref.at[j],
                    send_sem=asend_sems.at[j],
                    recv_sem=arecv_sems.at[j],
                    device_id=(j,),
                    device_id_type=pl.DeviceIdType.MESH,
                )
                recv.wait_recv()

        gmax = jnp.max(amax_ref[:, :, :])
        scale = gmax / 127.0
        yq = jnp.clip(jnp.round(y_ref[:, :] / scale), 0.0, 127.0)
        out_ref[:, :] = yq * scale

        for j in range(N_DEV):
            @pl.when(my != j)
            def _(j=j):
                s1 = pltpu.make_async_remote_copy(
                    src_ref=x_ref.at[pl.ds(0, BLK), :],
                    dst_ref=xg_ref.at[:, pl.ds(0, BLK)],
                    send_sem=send_sems.at[j],
                    recv_sem=recv_sems.at[j],
                    device_id=(j,),
                    device_id_type=pl.DeviceIdType.MESH,
                )
                s1.wait_send()
                s2 = pltpu.make_async_remote_copy(
                    src_ref=amax_ref.at[my],
                    dst_ref=amax_ref.at[my],
                    send_sem=asend_sems.at[j],
                    recv_sem=arecv_sems.at[j],
                    device_id=(j,),
                    device_id_type=pl.DeviceIdType.MESH,
                )
                s2.wait_send()

    return pl.pallas_call(
        body,
        out_shape=jax.ShapeDtypeStruct((BLK, n), jnp.float32),
        in_specs=[
            pl.BlockSpec(memory_space=pltpu.VMEM),
            pl.BlockSpec(memory_space=pltpu.VMEM),
        ],
        out_specs=pl.BlockSpec(memory_space=pltpu.VMEM),
        scratch_shapes=[
            pltpu.VMEM((BLK, k), jnp.float32),
            pltpu.VMEM((BLK, n), jnp.float32),
            pltpu.VMEM((N_DEV, 8, 128), jnp.float32),
            pltpu.SemaphoreType.DMA((N_DEV,)),
            pltpu.SemaphoreType.DMA((N_DEV,)),
            pltpu.SemaphoreType.DMA((N_DEV,)),
            pltpu.SemaphoreType.DMA((N_DEV,)),
        ],
        compiler_params=pltpu.CompilerParams(collective_id=0),
    )(x, w_mat)


# baseline (device time: 81099 ns/iter reference)
import jax
import jax.numpy as jnp
from jax import lax
from jax.experimental import pallas as pl
from jax.experimental.pallas import tpu as pltpu

N_DEV = 32
BLK = 128


def kernel(x, w_mat):
    k, m_per = x.shape
    _, n = w_mat.shape

    def body(x_ref, w_ref, out_ref, xg_ref, y_ref, amax_ref,
             send_sems, recv_sems, asend_sems, arecv_sems):
        my = lax.axis_index("i")

        for j in range(N_DEV):
            @pl.when(my != j)
            def _(j=j):
                rdma = pltpu.make_async_remote_copy(
                    src_ref=x_ref.at[pl.ds(j * BLK, BLK), :],
                    dst_ref=xg_ref.at[:, pl.ds(my * BLK, BLK)],
                    send_sem=send_sems.at[j],
                    recv_sem=recv_sems.at[my],
                    device_id=(j,),
                    device_id_type=pl.DeviceIdType.MESH,
                )
                rdma.start()

        xg_ref[:, pl.ds(my * BLK, BLK)] = x_ref[pl.ds(my * BLK, BLK), :]

        for j in range(N_DEV):
            @pl.when(my != j)
            def _(j=j):
                recv = pltpu.make_async_remote_copy(
                    src_ref=x_ref.at[pl.ds(0, BLK), :],
                    dst_ref=xg_ref.at[:, pl.ds(j * BLK, BLK)],
                    send_sem=send_sems.at[j],
                    recv_sem=recv_sems.at[j],
                    device_id=(j,),
                    device_id_type=pl.DeviceIdType.MESH,
                )
                recv.wait_recv()

        y = jnp.dot(
            xg_ref[:, :], w_ref[:, :],
            preferred_element_type=jnp.float32,
            precision=lax.Precision.HIGHEST,
        )
        y = jnp.maximum(y, 0.0)
        y_ref[:, :] = y
        amax_ref[my, :, :] = jnp.full((8, 128), jnp.max(y), jnp.float32)

        for j in range(N_DEV):
            @pl.when(my != j)
            def _(j=j):
                rdma = pltpu.make_async_remote_copy(
                    src_ref=amax_ref.at[my],
                    dst_ref=amax_ref.at[my],
                    send_sem=asend_sems.at[j],
                    recv_sem=arecv_sems.at[my],
                    device_id=(j,),
                    device_id_type=pl.DeviceIdType.MESH,
                )
                rdma.start()
        for j in range(N_DEV):
            @pl.when(my != j)
            def _(j=j):
                recv = pltpu.make_async_remote_copy(
                    src_ref=amax_ref.at[my],
                    dst_ref=amax_ref.at[j],
                    send_sem=asend_sems.at[j],
                    recv_sem=arecv_sems.at[j],
                    device_id=(j,),
                    device_id_type=pl.DeviceIdType.MESH,
                )
                recv.wait_recv()

        gmax = jnp.max(amax_ref[:, :, :])
        scale = gmax / 127.0
        yq = jnp.clip(jnp.round(y_ref[:, :] / scale), 0.0, 127.0)
        out_ref[:, :] = yq * scale

        for j in range(N_DEV):
            @pl.when(my != j)
            def _(j=j):
                s1 = pltpu.make_async_remote_copy(
                    src_ref=x_ref.at[pl.ds(0, BLK), :],
                    dst_ref=xg_ref.at[:, pl.ds(0, BLK)],
                    send_sem=send_sems.at[j],
                    recv_sem=recv_sems.at[j],
                    device_id=(j,),
                    device_id_type=pl.DeviceIdType.MESH,
                )
                s1.wait_send()
                s2 = pltpu.make_async_remote_copy(
                    src_ref=amax_ref.at[my],
                    dst_ref=amax_ref.at[my],
                    send_sem=asend_sems.at[j],
                    recv_sem=arecv_sems.at[j],
                    device_id=(j,),
                    device_id_type=pl.DeviceIdType.MESH,
                )
                s2.wait_send()

    return pl.pallas_call(
        body,
        out_shape=jax.ShapeDtypeStruct((BLK, n), jnp.float32),
        in_specs=[
            pl.BlockSpec(memory_space=pltpu.VMEM),
            pl.BlockSpec(memory_space=pltpu.VMEM),
        ],
        out_specs=pl.BlockSpec(memory_space=pltpu.VMEM),
        scratch_shapes=[
            pltpu.VMEM((BLK, k), jnp.float32),
            pltpu.VMEM((BLK, n), jnp.float32),
            pltpu.VMEM((N_DEV, 8, 128), jnp.float32),
            pltpu.SemaphoreType.DMA((N_DEV,)),
            pltpu.SemaphoreType.DMA((N_DEV,)),
            pltpu.SemaphoreType.DMA((N_DEV,)),
            pltpu.SemaphoreType.DMA((N_DEV,)),
        ],
        compiler_params=pltpu.CompilerParams(
            vmem_limit_bytes=60 * 1024 * 1024,
        ),
    )(x, w_mat)


# device time: 50922 ns/iter; 1.5926x vs baseline; 1.5926x over previous
import jax
import jax.numpy as jnp
from jax import lax
from jax.experimental import pallas as pl
from jax.experimental.pallas import tpu as pltpu

N_DEV = 32
BLK = 128
CHUNK = 4
N_CHUNKS = N_DEV // CHUNK
KC = CHUNK * BLK


def _split3(v):
    hi = v.astype(jnp.bfloat16)
    lo = (v - hi.astype(jnp.float32)).astype(jnp.bfloat16)
    return hi, lo


def kernel(x, w_mat):
    k, m_per = x.shape
    n = w_mat.shape[1]

    def body(x_ref, w_hbm, out_ref, xg_ref, y_ref, wbuf_ref, amax_ref,
             xbf_ref, send_sems, recv_sems, asend_sems, arecv_sems, wsems,
             bsems):
        my = lax.axis_index("i")
        c0 = my // CHUNK

        pz = my // 8
        pp = my % 8
        py = pp // 2
        px = jnp.where(py % 2 == 0, pp % 2, 1 - pp % 2)

        def to_idx(x_, y_, z_):
            return z_ * 8 + 2 * y_ + jnp.where(y_ % 2 == 0, x_, 1 - x_)

        partners = [
            to_idx(1 - px, py, pz),
            to_idx(px, py ^ 1, pz),
            to_idx(px, py ^ 2, pz),
            to_idx(px, py, pz ^ 1),
            to_idx(px, py, pz ^ 2),
        ]

        def w_dma(t, buf):
            c = lax.rem(c0 + t, N_CHUNKS)
            return pltpu.make_async_copy(
                w_hbm.at[pl.ds(c * KC, KC), :], wbuf_ref.at[buf],
                wsems.at[buf],
            )

        w_dma(0, 0).start()
        xbf_ref[:, :] = x_ref[:, :].astype(jnp.bfloat16)
        xg_ref[:, pl.ds(my * BLK, BLK)] = xbf_ref[pl.ds(my * BLK, BLK), :]

        barrier_sem = pltpu.get_barrier_semaphore()

        def barrier_round(r):
            sem = barrier_sem if r == 0 else bsems.at[r]
            pl.semaphore_signal(
                sem, inc=1,
                device_id=(partners[r],), device_id_type=pl.DeviceIdType.MESH,
            )
            pl.semaphore_wait(sem, 1)

        sidx = [0]

        def send_to(bx, by, bz):
            sidx[0] += 1
            jt = to_idx(px ^ bx, py ^ by, pz ^ bz)
            rdma = pltpu.make_async_remote_copy(
                src_ref=xbf_ref.at[pl.ds(jt * BLK, BLK), :],
                dst_ref=xg_ref.at[:, pl.ds(my * BLK, BLK)],
                send_sem=send_sems.at[sidx[0]],
                recv_sem=recv_sems.at[my],
                device_id=(jt,),
                device_id_type=pl.DeviceIdType.MESH,
            )
            rdma.start()

        plane_offs = [(1, 0), (0, 1), (1, 1), (0, 2), (1, 2), (0, 3), (1, 3)]

        barrier_round(0)
        barrier_round(1)
        barrier_round(2)
        for bx, by in plane_offs:
            send_to(bx, by, 0)
        barrier_round(3)
        for bx, by in [(0, 0)] + plane_offs:
            send_to(bx, by, 1)
        barrier_round(4)
        for bz in (2, 3):
            for bx, by in [(0, 0)] + plane_offs:
                send_to(bx, by, bz)

        w_dma(1, 1).start()

        for t in range(N_CHUNKS):
            c = lax.rem(c0 + t, N_CHUNKS)
            for u in range(CHUNK):
                j = c * CHUNK + u
                @pl.when(j != my)
                def _(j=j):
                    recv = pltpu.make_async_remote_copy(
                        src_ref=xbf_ref.at[pl.ds(0, BLK), :],
                        dst_ref=xg_ref.at[:, pl.ds(j * BLK, BLK)],
                        send_sem=send_sems.at[0],
                        recv_sem=recv_sems.at[j],
                        device_id=(0,),
                        device_id_type=pl.DeviceIdType.MESH,
                    )
                    recv.wait_recv()
            w_dma(t, t % 2).wait()
            if t + 1 < N_CHUNKS:
                if t >= 1:
                    w_dma(t + 1, (t + 1) % 2).start()
            xh = xg_ref[:, pl.ds(c * KC, KC)]
            wc = wbuf_ref[t % 2]
            wh, wl = _split3(wc)
            acc = jnp.dot(xh, wh, preferred_element_type=jnp.float32)
            acc += jnp.dot(xh, wl, preferred_element_type=jnp.float32)
            if t == 0:
                y_ref[:, :] = acc
            else:
                y_ref[:, :] = y_ref[:, :] + acc

        lmax = jnp.maximum(jnp.max(y_ref[:, :]), 0.0)
        amax_ref[0, :, :] = jnp.full((8, 128), lmax, jnp.float32)
        for r, prt in enumerate(partners):
            rdma = pltpu.make_async_remote_copy(
                src_ref=amax_ref.at[0],
                dst_ref=amax_ref.at[1 + r],
                send_sem=asend_sems.at[r],
                recv_sem=arecv_sems.at[r],
                device_id=(prt,),
                device_id_type=pl.DeviceIdType.MESH,
            )
            rdma.start()
            rdma.wait()
            amax_ref[0, :, :] = jnp.maximum(
                amax_ref[0, :, :], amax_ref[1 + r, :, :]
            )

        gmax = jnp.max(amax_ref[0, :, :])
        scale = gmax / 127.0
        yr = jnp.maximum(y_ref[:, :], 0.0)
        yq = jnp.clip(jnp.round(yr / scale), 0.0, 127.0)
        out_ref[:, :] = yq * scale

        for dj in range(1, N_DEV):
            s1 = pltpu.make_async_remote_copy(
                src_ref=xbf_ref.at[pl.ds(0, BLK), :],
                dst_ref=xg_ref.at[:, pl.ds(0, BLK)],
                send_sem=send_sems.at[dj],
                recv_sem=recv_sems.at[0],
                device_id=(0,),
                device_id_type=pl.DeviceIdType.MESH,
            )
            s1.wait_send()

    return pl.pallas_call(
        body,
        out_shape=jax.ShapeDtypeStruct((BLK, n), jnp.float32),
        in_specs=[
            pl.BlockSpec(memory_space=pltpu.VMEM),
            pl.BlockSpec(memory_space=pl.ANY),
        ],
        out_specs=pl.BlockSpec(memory_space=pltpu.VMEM),
        scratch_shapes=[
            pltpu.VMEM((BLK, k), jnp.bfloat16),
            pltpu.VMEM((BLK, n), jnp.float32),
            pltpu.VMEM((2, KC, n), jnp.float32),
            pltpu.VMEM((6, 8, 128), jnp.float32),
            pltpu.VMEM((k, BLK), jnp.bfloat16),
            pltpu.SemaphoreType.DMA((N_DEV,)),
            pltpu.SemaphoreType.DMA((N_DEV,)),
            pltpu.SemaphoreType.DMA((5,)),
            pltpu.SemaphoreType.DMA((5,)),
            pltpu.SemaphoreType.DMA((2,)),
            pltpu.SemaphoreType.REGULAR((5,)),
        ],
        compiler_params=pltpu.CompilerParams(
            vmem_limit_bytes=60 * 1024 * 1024,
            collective_id=0,
        ),
    )(x, w_mat)
